# Initial kernel scaffold; baseline (speedup 1.0000x reference)
import jax
import jax.numpy as jnp
from jax.experimental import pallas as pl


def kernel(x, edge_index, batch, x_e, edge_index_e, W1, b1, W2, b2, W3, b3, W4, b4):
    raise NotImplementedError("write your pallas kernel here")


# speedup vs baseline: 10.7495x; 10.7495x over previous
import functools

import jax
import jax.numpy as jnp
from jax import lax
from jax.experimental import pallas as pl
from jax.experimental.pallas import tpu as pltpu
from jax.experimental.pallas import tpu_sc as plsc

N1, E1 = 50000, 800000
N2, E2 = 10000, 160000
G = 512

NC, NS, L = 2, 16, 16
NW = NC * NS
BLK = 128
PADV = 1 << 20

EP1 = 802816
EP2 = 163840
EPB = 57344

f32 = jnp.float32
i32 = jnp.int32


def _mesh():
    return plsc.VectorSubcoreMesh(
        core_axis_name="c", subcore_axis_name="s", num_cores=NC, num_subcores=NS
    )


def _zero_vmem(buf, nrows, d):
    nv = d // L

    def body(t, _):
        r = t // nv
        j = t - r * nv
        buf[r, pl.ds(j * L, L)] = jnp.zeros((L,), f32)
        return 0

    lax.fori_loop(0, nrows * nv, body, 0)


def _zero_shared(acc, zbuf, sid, z_per_tile, zr):
    def body(k, _):
        pltpu.sync_copy(zbuf, acc.at[pl.ds(sid * z_per_tile + k * zr, zr)])
        return 0

    lax.fori_loop(0, z_per_tile // zr, body, 0)


def _transform_dst(didx, buf, base, half, dump_base):
    for k in range(BLK // L):
        d = didx[buf, pl.ds(k * L, L)]
        owned = jnp.logical_and(d >= base, d < base + half)
        dump = dump_base + lax.iota(i32, L)
        didx[buf, pl.ds(k * L, L)] = jnp.where(owned, d - base, dump)


def _writeback(acc, out_hbm, sid, core, half, cr):
    t_chunks = half // cr

    def body(k, _):
        q = sid + k * NS
        pltpu.sync_copy(
            acc.at[pl.ds(q * cr, cr)], out_hbm.at[pl.ds(core * half + q * cr, cr)]
        )
        return 0

    lax.fori_loop(0, (t_chunks - 1 - sid) // NS + 1, body, 0)


def _make_propagate(n, d, e_pad, r_rows, zr, cr, interpret=False):
    half = n // 2
    nblk = e_pad // (NS * BLK)
    assert nblk % 2 == 0
    z_per_tile = r_rows // NS

    @functools.partial(
        pl.kernel,
        out_type=jax.ShapeDtypeStruct((n, d), f32),
        mesh=_mesh(),
        scratch_types=[
            pltpu.VMEM((2, BLK), i32),
            pltpu.VMEM((2, BLK), i32),
            pltpu.VMEM((2, BLK, d), f32),
            pltpu.VMEM((zr, d), f32),
            pltpu.VMEM_SHARED((r_rows, d), f32),
            pltpu.SemaphoreType.DMA,
            pltpu.SemaphoreType.DMA,
        ],
        name=f"gcn_propagate_{n}_{d}",
        compiler_params=pltpu.CompilerParams(use_tc_tiling_on_sc=False),
        interpret=interpret,
    )
    def prop(g_hbm, src_hbm, dst_hbm, out_hbm, sidx, didx, rows, zbuf, acc, sem0, sem1):
        core = lax.axis_index("c")
        sid = lax.axis_index("s")
        base = core * half
        dump_base = half + sid * L

        _zero_vmem(zbuf, zr, d)
        _zero_shared(acc, zbuf, sid, z_per_tile, zr)
        plsc.subcore_barrier()

        def load_ids(b, buf):
            off = (sid * nblk + b) * BLK
            pltpu.sync_copy(src_hbm.at[pl.ds(off, BLK)], sidx.at[buf])
            pltpu.sync_copy(dst_hbm.at[pl.ds(off, BLK)], didx.at[buf])
            _transform_dst(didx, buf, base, half, dump_base)

        def scatter(buf):
            pltpu.sync_copy(rows.at[buf], acc.at[didx.at[buf]], add=True)

        def body(it, _):
            load_ids(2 * it, 0)
            d0 = pltpu.async_copy(g_hbm.at[sidx.at[0]], rows.at[0], sem0)

            @pl.when(it > 0)
            def _():
                pltpu.make_async_copy(g_hbm.at[sidx.at[1]], rows.at[1], sem1).wait()
                scatter(1)

            load_ids(2 * it + 1, 1)
            pltpu.async_copy(g_hbm.at[sidx.at[1]], rows.at[1], sem1)
            d0.wait()
            scatter(0)
            return 0

        lax.fori_loop(0, nblk // 2, body, 0)
        pltpu.make_async_copy(g_hbm.at[sidx.at[1]], rows.at[1], sem1).wait()
        scatter(1)

        plsc.subcore_barrier()
        _writeback(acc, out_hbm, sid, core, half, cr)

    return prop


_prop_a = _make_propagate(N1, 64, EP1, 25600, 50, 200)
_prop_b128 = _make_propagate(N2, 128, EP2, 5600, 50, 200)
_prop_b64 = _make_propagate(N2, 64, EP2, 5600, 50, 200)


_CNT_PHASES = (
    (N1, 25000, 25600, EP1, 200),
    (N2, 5000, 5376, EP2, 200),
    (G, 256, 768, EPB, 16),
)


def _build_counts(interpret=False):
  return functools.partial(
    pl.kernel,
    out_type=[
        jax.ShapeDtypeStruct((N1, L), f32),
        jax.ShapeDtypeStruct((N2, L), f32),
        jax.ShapeDtypeStruct((G, L), f32),
    ],
    mesh=_mesh(),
    scratch_types=[
        pltpu.VMEM((1, BLK), i32),
        pltpu.VMEM((BLK, L), f32),
        pltpu.VMEM((16, L), f32),
        pltpu.VMEM_SHARED((25600, L), f32),
        pltpu.VMEM_SHARED((5376, L), f32),
        pltpu.VMEM_SHARED((768, L), f32),
    ],
    name="gcn_counts",
    compiler_params=pltpu.CompilerParams(use_tc_tiling_on_sc=False),
    interpret=interpret,
  )(_counts_body)


def _counts_body(dst1, dst2, batchp, deg1w, deg2w, histw, didx, ones, zbuf, acc1, acc2, acc3):
    core = lax.axis_index("c")
    sid = lax.axis_index("s")

    _zero_vmem(zbuf, 16, L)
    for acc, (_, _, r_rows, _, _) in zip((acc1, acc2, acc3), _CNT_PHASES):
        _zero_shared(acc, zbuf, sid, r_rows // NS, 16)

    def fill_ones(t, _):
        ones[t, pl.ds(0, L)] = jnp.ones((L,), f32)
        return 0

    lax.fori_loop(0, BLK, fill_ones, 0)
    plsc.subcore_barrier()

    for acc, dst_hbm, (_, half, _, e_pad, _) in zip(
        (acc1, acc2, acc3), (dst1, dst2, batchp), _CNT_PHASES
    ):
        nblk = e_pad // (NS * BLK)
        base = core * half
        dump_base = half + sid * L

        def body(b, _, acc=acc, dst_hbm=dst_hbm, nblk=nblk, base=base,
                 half=half, dump_base=dump_base):
            off = (sid * nblk + b) * BLK
            pltpu.sync_copy(dst_hbm.at[pl.ds(off, BLK)], didx.at[0])
            _transform_dst(didx, 0, base, half, dump_base)
            pltpu.sync_copy(ones, acc.at[didx.at[0]], add=True)
            return 0

        lax.fori_loop(0, nblk, body, 0)

    plsc.subcore_barrier()
    for acc, out_hbm, (_, half, _, _, cr) in zip(
        (acc1, acc2, acc3), (deg1w, deg2w, histw), _CNT_PHASES
    ):
        _writeback(acc, out_hbm, sid, core, half, cr)


N1P = 50176
SEG_PER_W = G // NW
CHK = 128


def _build_segmax(interpret=False):
  return functools.partial(
    pl.kernel,
    out_type=jax.ShapeDtypeStruct((G, 64), f32),
    mesh=_mesh(),
    scratch_types=[
        pltpu.VMEM((1024,), i32),
        pltpu.VMEM((CHK, 64), f32),
        pltpu.VMEM((SEG_PER_W, 64), f32),
    ],
    name="gcn_segmax",
    compiler_params=pltpu.CompilerParams(use_tc_tiling_on_sc=False),
    interpret=interpret,
  )(_segmax_body)


def _segmax_body(h_hbm, starts_hbm, xg_hbm, sbuf, cbuf, obuf):
    core = lax.axis_index("c")
    sid = lax.axis_index("s")
    wid = sid * NC + core
    pltpu.sync_copy(starts_hbm, sbuf)
    neg = jnp.full((L,), -jnp.inf, f32)
    v0 = sbuf[pl.ds(pl.multiple_of(wid * L, L), L)]
    v1 = sbuf[pl.ds(pl.multiple_of(wid * L + L, L), L)]

    for k in range(SEG_PER_W):
        s = jnp.clip(v0[k], 0, N1)
        e = jnp.clip(v1[0] if k == SEG_PER_W - 1 else v0[k + 1], s, N1)

        a0 = (s // 8) * 8

        def chunk_body(q, accs, s=s, e=e, a0=a0):
            pos = a0 + q * CHK
            pltpu.sync_copy(h_hbm.at[pl.ds(pos, CHK)], cbuf)
            lo = jnp.maximum(s - pos, 0)
            m = jnp.maximum(jnp.minimum(e - pos, CHK), lo)

            def row_body(rr, accs):
                return tuple(
                    jnp.maximum(accs[j], cbuf[rr, pl.ds(j * L, L)]) for j in range(4)
                )

            return lax.fori_loop(lo, m, row_body, accs)

        nch = lax.div(e - a0 + (CHK - 1), CHK)
        accs = lax.fori_loop(0, nch, chunk_body, (neg, neg, neg, neg))
        for j in range(4):
            obuf[k, pl.ds(j * L, L)] = accs[j]

    pltpu.sync_copy(obuf, xg_hbm.at[pl.ds(wid * SEG_PER_W, SEG_PER_W)])


_counts = _build_counts()
_segmax = _build_segmax()


def _dinv(deg_ref):
    return lax.rsqrt(deg_ref[:, :1] + 1.0)


def _tc_call(body, grid, in_specs, out_specs, out_shape, name):
    return pl.pallas_call(
        body, grid=grid, in_specs=in_specs, out_specs=out_specs,
        out_shape=out_shape, name=name,
    )


def _pre_body(x_ref, w_ref, deg_ref, o_ref):
    o_ref[...] = _dinv(deg_ref) * jnp.dot(
        x_ref[...], w_ref[...], preferred_element_type=f32
    )


def _mid_body(s_ref, g_ref, deg_ref, w_ref, b_ref, o_ref):
    dv = _dinv(deg_ref)
    h = jax.nn.relu(dv * (s_ref[...] + g_ref[...]) + b_ref[...])
    o_ref[...] = dv * jnp.dot(h, w_ref[...], preferred_element_type=f32)


def _post_body(s_ref, g_ref, deg_ref, b_ref, o_ref):
    o_ref[...] = jax.nn.relu(
        _dinv(deg_ref) * (s_ref[...] + g_ref[...]) + b_ref[...]
    )


def _starts_body(hist_ref, o_ref):
    hist = hist_ref[:, :1]
    row = lax.broadcasted_iota(i32, (1024, G), 0)
    col = lax.broadcasted_iota(i32, (1024, G), 1)
    mask = jnp.where(col < row, 1.0, 0.0).astype(f32)
    st = jnp.dot(mask, hist, preferred_element_type=f32)
    o_ref[...] = st[:, 0].astype(i32)


def _final_body(xg_ref, he_ref, o_ref):
    o_ref[...] = lax.dot_general(
        xg_ref[...], he_ref[...], (((1,), (1,)), ((), ())),
        preferred_element_type=f32,
    )


def _row_spec(rb, d):
    return pl.BlockSpec((rb, d), lambda i: (i, 0))


def _full_spec(shape):
    return pl.BlockSpec(shape, lambda i: tuple(0 for _ in shape))


def _make_pre(n, din, dout, rb):
    return _tc_call(
        _pre_body, (n // rb,),
        [_row_spec(rb, din), _full_spec((din, dout)), _row_spec(rb, L)],
        _row_spec(rb, dout), jax.ShapeDtypeStruct((n, dout), f32), "gcn_pre",
    )


def _make_mid(n, d, dout, rb):
    return _tc_call(
        _mid_body, (n // rb,),
        [_row_spec(rb, d), _row_spec(rb, d), _row_spec(rb, L),
         _full_spec((d, dout)), _full_spec((d,))],
        _row_spec(rb, dout), jax.ShapeDtypeStruct((n, dout), f32), "gcn_mid",
    )


def _make_post(n, d, rb, n_out=None):
    return _tc_call(
        _post_body, (n // rb,),
        [_row_spec(rb, d), _row_spec(rb, d), _row_spec(rb, L), _full_spec((d,))],
        _row_spec(rb, d), jax.ShapeDtypeStruct((n_out or n, d), f32), "gcn_post",
    )


_pre_a = _make_pre(N1, 78, 64, 1000)
_mid_a = _make_mid(N1, 64, 64, 1000)
_post_a = _make_post(N1, 64, 1000, N1P)
_pre_b = _make_pre(N2, 243, 128, 1000)
_mid_b = _make_mid(N2, 128, 64, 1000)
_post_b = _make_post(N2, 64, 1000)

_starts_tc = _tc_call(
    _starts_body, (1,), [_full_spec((G, L))], _full_spec((1024,)),
    jax.ShapeDtypeStruct((1024,), i32), "gcn_starts",
)

_final_tc = _tc_call(
    _final_body, (1,),
    [_full_spec((G, 64)), _full_spec((N2, 64))],
    _full_spec((G, N2)),
    jax.ShapeDtypeStruct((G, N2), f32), "gcn_final",
)


def _pad_edges(edge_index, e, e_pad):
    src = jnp.concatenate(
        [edge_index[0].astype(i32), jnp.zeros((e_pad - e,), i32)]
    )
    dst = jnp.concatenate(
        [edge_index[1].astype(i32), jnp.full((e_pad - e,), PADV, i32)]
    )
    return src, dst


@jax.jit
def _run(x, edge_index, batch, x_e, edge_index_e, W1, b1, W2, b2, W3, b3, W4, b4):
    src1, dst1 = _pad_edges(edge_index, E1, EP1)
    src2, dst2 = _pad_edges(edge_index_e, E2, EP2)
    batchp = jnp.concatenate(
        [batch.astype(i32), jnp.full((EPB - N1,), PADV, i32)]
    )

    deg1w, deg2w, histw = _counts(dst1, dst2, batchp)
    starts = _starts_tc(histw)

    g1 = _pre_a(x, W1, deg1w)
    s1 = _prop_a(g1, src1, dst1)
    g2 = _mid_a(s1, g1, deg1w, W2, b1)
    s2 = _prop_a(g2, src1, dst1)
    h2p = _post_a(s2, g2, deg1w, b2)
    xg = _segmax(h2p, starts)

    g3 = _pre_b(x_e, W3, deg2w)
    s3 = _prop_b128(g3, src2, dst2)
    g4 = _mid_b(s3, g3, deg2w, W4, b3)
    s4 = _prop_b64(g4, src2, dst2)
    he = _post_b(s4, g4, deg2w, b4)

    xc = _final_tc(xg, he)
    return xc, xg, he


def kernel(x, edge_index, batch, x_e, edge_index_e, W1, b1, W2, b2, W3, b3, W4, b4):
    return _run(x, edge_index, batch, x_e, edge_index_e,
                W1, b1, W2, b2, W3, b3, W4, b4)
